# baseline (speedup 1.0000x reference)
import jax
import jax.numpy as jnp
import numpy as np
from jax import lax
from jax.experimental import pallas as pl
from jax.experimental.pallas import tpu as pltpu
from jax.experimental.pallas import tpu_sc as plsc

D = 3
L = 16
C = 2
BASE = 16.0
LOG2_HASHMAP = 19
B = 524288
PRIMES = (1, 2654435761, 805459861)

NC = 2
NS = 16
NW = NC * NS
PT = B // NW
CH = 128
NCH = PT // CH
NG = CH // 16

HASH_PARAMS = 2 ** LOG2_HASHMAP
HASH_MASK = HASH_PARAMS - 1
_P0, _P1, _P2 = 4096, 32768, 262144
OFF3 = _P0 + _P1 + _P2
SLH = HASH_PARAMS * C // NS

_GRAY_ORDER = (0, 1, 3, 2, 6, 7, 5, 4)
_GRAY_FLIP = (0, 1, 0, 2, 0, 1, 0)


def _i32(x):
    return jnp.int32(np.uint32(x).astype(np.int32))


def _body(inp, emb, out, oacc, shr, xyz0, xyz1, fr0, fr1, idx0, idx1,
          rows0, rows1, sg0, sg1, si0, si1):
    cid = lax.axis_index("c")
    sid = lax.axis_index("s")
    wid = sid * NC + cid
    tb = wid * PT

    def in_descs(c, xyzb, sem):
        off = (tb + jnp.minimum(c, NCH - 1) * CH) * D
        return [pltpu.make_async_copy(inp.at[pl.ds(off, CH * D)], xyzb, sem)]

    def fire_inputs(c, xyzb, sem):
        for dsc in in_descs(c, xyzb, sem):
            dsc.start()

    def wait_inputs(c, xyzb, sem):
        for dsc in in_descs(c, xyzb, sem):
            dsc.wait()

    def gdesc(idxb, rowsb, sem):
        return pltpu.make_async_copy(shr.at[idxb], rowsb, sem)

    def index_phase(xyzb, frb, idxb, rowsb, sem, scale, use_hash, r=0):
        iota = lax.iota(jnp.int32, 16)

        def grp(g, _):
            s = pl.ds(g * 16, 16)
            i3 = (g * 16 + iota) * D
            x = plsc.load_gather(xyzb, [i3])
            y = plsc.load_gather(xyzb, [i3 + 1])
            z = plsc.load_gather(xyzb, [i3 + 2])
            px = x * scale + 0.5
            py = y * scale + 0.5
            pz = z * scale + 0.5
            gx = px.astype(jnp.int32)
            gy = py.astype(jnp.int32)
            gz = pz.astype(jnp.int32)
            frb[0, s] = px - gx.astype(jnp.float32)
            frb[1, s] = py - gy.astype(jnp.float32)
            frb[2, s] = pz - gz.astype(jnp.float32)
            cs = [None] * 8
            if use_hash:
                mask = jnp.int32(HASH_MASK)
                p1 = _i32(PRIMES[1])
                p2 = _i32(PRIMES[2])
                hy0 = gy * p1
                hz0 = gz * p2
                dh = (gx ^ (gx + 1), hy0 ^ (hy0 + p1), hz0 ^ (hz0 + p2))
                h = gx ^ hy0 ^ hz0
                cs[0] = h
                for step in range(7):
                    h = h ^ dh[_GRAY_FLIP[step]]
                    cs[_GRAY_ORDER[step + 1]] = h
            else:
                mask = jnp.int32(r * r * r - 1)
                ri = jnp.int32(r)
                r2 = jnp.int32(r * r)
                base = gx + gy * ri + gz * r2
                for k in range(8):
                    cur = base
                    if k & 1:
                        cur = cur + 1
                    if k & 2:
                        cur = cur + ri
                    if k & 4:
                        cur = cur + r2
                    cs[k] = cur
            for k in range(8):
                e0 = lax.shift_left(cs[k] & mask, 1)
                idxb[pl.ds(k * C * CH + g * 16, 16)] = e0
                idxb[pl.ds(k * C * CH + CH + g * 16, 16)] = e0 + 1
            return 0

        lax.fori_loop(0, NG, grp, 0, unroll=False)
        gdesc(idxb, rowsb, sem).start()

    def accum_phase(cc, frb, idxb, rowsb, sem):
        gdesc(idxb, rowsb, sem).wait()

        def grp(g, _):
            s = pl.ds(g * 16, 16)
            so = pl.ds(cc * CH + g * 16, 16)
            fx = frb[0, s]
            fy = frb[1, s]
            fz = frb[2, s]
            for ch in range(C):
                e = [rowsb[pl.ds(k * C * CH + ch * CH + g * 16, 16)]
                     for k in range(8)]
                ax = [e[m] + fx * (e[m + 1] - e[m]) for m in (0, 2, 4, 6)]
                ay = [ax[m] + fy * (ax[m + 1] - ax[m]) for m in (0, 2)]
                oacc[ch, so] = ay[0] + fz * (ay[1] - ay[0])
            return 0

        lax.fori_loop(0, NG, grp, 0, unroll=False)

    def run_level(lidx, src_off, sl, scale, use_hash, r=0):
        plsc.subcore_barrier()
        pltpu.sync_copy(emb.at[pl.ds(src_off + sid * sl, sl)],
                        shr.at[pl.ds(sid * sl, sl)])
        plsc.subcore_barrier()

        def idx_p(xyzb, frb, idxb, rowsb, sem):
            index_phase(xyzb, frb, idxb, rowsb, sem, scale, use_hash, r)

        fire_inputs(0, xyz0, si0)
        fire_inputs(1, xyz1, si1)
        wait_inputs(0, xyz0, si0)
        idx_p(xyz0, fr0, idx0, rows0, sg0)
        fire_inputs(2, xyz0, si0)

        def step(j, _):
            a = 2 * j + 1
            b = a + 1
            wait_inputs(a, xyz1, si1)
            idx_p(xyz1, fr1, idx1, rows1, sg1)
            fire_inputs(a + 2, xyz1, si1)
            accum_phase(a - 1, fr0, idx0, rows0, sg0)
            wait_inputs(b, xyz0, si0)
            idx_p(xyz0, fr0, idx0, rows0, sg0)
            fire_inputs(b + 2, xyz0, si0)
            accum_phase(a, fr1, idx1, rows1, sg1)
            return 0

        lax.fori_loop(0, NCH // 2, step, 0, unroll=False)
        gdesc(idx0, rows0, sg0).wait()
        wait_inputs(NCH + 1, xyz1, si1)
        wait_inputs(NCH + 2, xyz0, si0)

        for ch in range(C):
            pltpu.sync_copy(oacc.at[ch], out.at[lidx, ch, pl.ds(tb, PT)])

    run_level(0, 0, _P0 * C // NS, 15.0, False, 16)
    run_level(1, _P0 * C, _P1 * C // NS, 31.0, False, 32)
    run_level(2, (_P0 + _P1) * C, _P2 * C // NS, 63.0, False, 64)

    def hash_level(l, _):
        scale = (lax.shift_left(jnp.int32(16), l) - 1).astype(jnp.float32)
        src_off = OFF3 * C + (l - 3) * (HASH_PARAMS * C)
        run_level(l, src_off, SLH, scale, True)
        return 0

    lax.fori_loop(3, L, hash_level, 0, unroll=False)


_TBLK = 1024


def _tpose_body(in_ref, out_ref):
    out_ref[...] = in_ref[...].T


@jax.jit
def kernel(inputs, embeddings):
    inp = inputs.reshape(-1)
    emb = embeddings.reshape(-1)
    mesh = plsc.VectorSubcoreMesh(core_axis_name="c", subcore_axis_name="s",
                                  num_cores=NC, num_subcores=NS)
    out = pl.kernel(
        _body,
        out_type=jax.ShapeDtypeStruct((L, C, B), jnp.float32),
        mesh=mesh,
        compiler_params=pltpu.CompilerParams(needs_layout_passes=False,
                                             use_tc_tiling_on_sc=False),
        scratch_types=[
            pltpu.VMEM((C, PT), jnp.float32),
            pltpu.VMEM_SHARED((HASH_PARAMS * C,), jnp.float32),
            pltpu.VMEM((CH * D,), jnp.float32),
            pltpu.VMEM((CH * D,), jnp.float32),
            pltpu.VMEM((D, CH), jnp.float32),
            pltpu.VMEM((D, CH), jnp.float32),
            pltpu.VMEM((8 * C * CH,), jnp.int32),
            pltpu.VMEM((8 * C * CH,), jnp.int32),
            pltpu.VMEM((8 * C * CH,), jnp.float32),
            pltpu.VMEM((8 * C * CH,), jnp.float32),
            pltpu.SemaphoreType.DMA,
            pltpu.SemaphoreType.DMA,
            pltpu.SemaphoreType.DMA,
            pltpu.SemaphoreType.DMA,
        ],
    )(inp, emb)
    return pl.pallas_call(
        _tpose_body,
        grid=(B // _TBLK,),
        in_specs=[pl.BlockSpec((L * C, _TBLK), lambda i: (0, i))],
        out_specs=pl.BlockSpec((_TBLK, L * C), lambda i: (i, 0)),
        out_shape=jax.ShapeDtypeStruct((B, L * C), jnp.float32),
    )(out.reshape(L * C, B))


# speedup vs baseline: 6.6764x; 6.6764x over previous
import jax
import jax.numpy as jnp
import numpy as np
from jax import lax
from jax.experimental import pallas as pl
from jax.experimental.pallas import tpu as pltpu
from jax.experimental.pallas import tpu_sc as plsc

D = 3
L = 16
C = 2
BASE = 16.0
LOG2_HASHMAP = 19
B = 524288
PRIMES = (1, 2654435761, 805459861)

NC = 2
NS = 16
NW = NC * NS
PT = B // NW
CH = 128
NCH = PT // CH
NG = CH // 16

HASH_PARAMS = 2 ** LOG2_HASHMAP
HASH_MASK = HASH_PARAMS - 1
_P0, _P1, _P2 = 4096, 32768, 262144
OFF3 = _P0 + _P1 + _P2

_GRAY_ORDER = (0, 1, 3, 2, 6, 7, 5, 4)
_GRAY_FLIP = (0, 1, 0, 2, 0, 1, 0)


def _i32(x):
    return jnp.int32(np.uint32(x).astype(np.int32))


def _body(xs, ys, zs, emb0, emb1, out, oacc, shr, xyz0, xyz1, fr0, fr1,
          idx0, idx1, rows0, rows1, sg0, sg1, si0, si1):
    cid = lax.axis_index("c")
    sid = lax.axis_index("s")
    wid = sid * NC + cid
    tb = wid * PT

    def in_descs(c, xyzb, sem):
        off = tb + jnp.minimum(c, NCH - 1) * CH
        return [pltpu.make_async_copy(src.at[pl.ds(off, CH)], xyzb.at[d], sem)
                for d, src in enumerate((xs, ys, zs))]

    def fire_inputs(c, xyzb, sem):
        for dsc in in_descs(c, xyzb, sem):
            dsc.start()

    def wait_inputs(c, xyzb, sem):
        for dsc in in_descs(c, xyzb, sem):
            dsc.wait()

    def gdesc(idxb, rowsb, sem):
        return pltpu.make_async_copy(shr.at[idxb], rowsb, sem)

    def index_phase(xyzb, frb, idxb, rowsb, sem, scale, use_hash, prm, r=0):
        def grp(g, _):
            s = pl.ds(g * 16, 16)
            x = xyzb[0, s]
            y = xyzb[1, s]
            z = xyzb[2, s]
            px = x * scale + 0.5
            py = y * scale + 0.5
            pz = z * scale + 0.5
            gx = px.astype(jnp.int32)
            gy = py.astype(jnp.int32)
            gz = pz.astype(jnp.int32)
            frb[0, s] = px - gx.astype(jnp.float32)
            frb[1, s] = py - gy.astype(jnp.float32)
            frb[2, s] = pz - gz.astype(jnp.float32)
            cs = [None] * 8
            if use_hash:
                mask = jnp.int32(HASH_MASK)
                p1 = _i32(PRIMES[1])
                p2 = _i32(PRIMES[2])
                hy0 = gy * p1
                hz0 = gz * p2
                dh = (gx ^ (gx + 1), hy0 ^ (hy0 + p1), hz0 ^ (hz0 + p2))
                h = gx ^ hy0 ^ hz0
                cs[0] = h
                for step in range(7):
                    h = h ^ dh[_GRAY_FLIP[step]]
                    cs[_GRAY_ORDER[step + 1]] = h
            else:
                mask = jnp.int32(r * r * r - 1)
                ri = jnp.int32(r)
                r2 = jnp.int32(r * r)
                base = gx + gy * ri + gz * r2
                for k in range(8):
                    cur = base
                    if k & 1:
                        cur = cur + 1
                    if k & 2:
                        cur = cur + ri
                    if k & 4:
                        cur = cur + r2
                    cs[k] = cur
            for k in range(8):
                e0 = cs[k] & mask
                idxb[pl.ds(k * C * CH + g * 16, 16)] = e0
                idxb[pl.ds(k * C * CH + CH + g * 16, 16)] = e0 + prm
            return 0

        lax.fori_loop(0, NG, grp, 0, unroll=False)
        gdesc(idxb, rowsb, sem).start()

    def accum_phase(cc, frb, idxb, rowsb, sem):
        gdesc(idxb, rowsb, sem).wait()

        def grp(g, _):
            s = pl.ds(g * 16, 16)
            fx = frb[0, s]
            fy = frb[1, s]
            fz = frb[2, s]
            for ch in range(C):
                so = pl.ds(ch * PT + cc * CH + g * 16, 16)
                e = [rowsb[pl.ds(k * C * CH + ch * CH + g * 16, 16)]
                     for k in range(8)]
                ax = [e[m] + fx * (e[m + 1] - e[m]) for m in (0, 2, 4, 6)]
                ay = [ax[m] + fy * (ax[m + 1] - ax[m]) for m in (0, 2)]
                oacc[so] = ay[0] + fz * (ay[1] - ay[0])
            return 0

        lax.fori_loop(0, NG, grp, 0, unroll=False)

    def run_level(lidx, roff, prm, scale, use_hash, r=0):
        slr = prm // NS
        plsc.subcore_barrier()
        pltpu.sync_copy(emb0.at[pl.ds(roff + sid * slr, slr)],
                        shr.at[pl.ds(sid * slr, slr)])
        pltpu.sync_copy(emb1.at[pl.ds(roff + sid * slr, slr)],
                        shr.at[pl.ds(prm + sid * slr, slr)])
        plsc.subcore_barrier()
        prmi = jnp.int32(prm)

        def idx_p(xyzb, frb, idxb, rowsb, sem):
            index_phase(xyzb, frb, idxb, rowsb, sem, scale, use_hash, prmi, r)

        fire_inputs(0, xyz0, si0)
        fire_inputs(1, xyz1, si1)
        wait_inputs(0, xyz0, si0)
        idx_p(xyz0, fr0, idx0, rows0, sg0)
        fire_inputs(2, xyz0, si0)

        def step(j, _):
            a = 2 * j + 1
            b = a + 1
            wait_inputs(a, xyz1, si1)
            idx_p(xyz1, fr1, idx1, rows1, sg1)
            fire_inputs(a + 2, xyz1, si1)
            accum_phase(a - 1, fr0, idx0, rows0, sg0)
            wait_inputs(b, xyz0, si0)
            idx_p(xyz0, fr0, idx0, rows0, sg0)
            fire_inputs(b + 2, xyz0, si0)
            accum_phase(a, fr1, idx1, rows1, sg1)
            return 0

        lax.fori_loop(0, NCH // 2, step, 0, unroll=False)
        gdesc(idx0, rows0, sg0).wait()
        wait_inputs(NCH + 1, xyz1, si1)
        wait_inputs(NCH + 2, xyz0, si0)

        for ch in range(C):
            pltpu.sync_copy(oacc.at[pl.ds(ch * PT, PT)],
                            out.at[pl.ds((lidx * C + ch) * B + tb, PT)])

    run_level(0, 0, _P0, 15.0, False, 16)
    run_level(1, _P0, _P1, 31.0, False, 32)
    run_level(2, _P0 + _P1, _P2, 63.0, False, 64)

    def hash_level(l, _):
        scale = (lax.shift_left(jnp.int32(16), l) - 1).astype(jnp.float32)
        roff = OFF3 + (l - 3) * HASH_PARAMS
        run_level(l, roff, HASH_PARAMS, scale, True)
        return 0

    lax.fori_loop(3, L, hash_level, 0, unroll=False)


@jax.jit
def kernel(inputs, embeddings):
    xs, ys, zs = inputs[:, 0], inputs[:, 1], inputs[:, 2]
    emb0, emb1 = embeddings[:, 0], embeddings[:, 1]
    mesh = plsc.VectorSubcoreMesh(core_axis_name="c", subcore_axis_name="s",
                                  num_cores=NC, num_subcores=NS)
    out = pl.kernel(
        _body,
        out_type=jax.ShapeDtypeStruct((L * C * B,), jnp.float32),
        mesh=mesh,
        compiler_params=pltpu.CompilerParams(needs_layout_passes=False,
                                             use_tc_tiling_on_sc=False),
        scratch_types=[
            pltpu.VMEM((C * PT,), jnp.float32),
            pltpu.VMEM_SHARED((C * HASH_PARAMS,), jnp.float32),
            pltpu.VMEM((D, CH), jnp.float32),
            pltpu.VMEM((D, CH), jnp.float32),
            pltpu.VMEM((D, CH), jnp.float32),
            pltpu.VMEM((D, CH), jnp.float32),
            pltpu.VMEM((8 * C * CH,), jnp.int32),
            pltpu.VMEM((8 * C * CH,), jnp.int32),
            pltpu.VMEM((8 * C * CH,), jnp.float32),
            pltpu.VMEM((8 * C * CH,), jnp.float32),
            pltpu.SemaphoreType.DMA,
            pltpu.SemaphoreType.DMA,
            pltpu.SemaphoreType.DMA,
            pltpu.SemaphoreType.DMA,
        ],
    )(xs, ys, zs, emb0, emb1)
    return out.reshape(L, C, B).transpose(2, 0, 1).reshape(B, L * C)
